# baseline (speedup 1.0000x reference)
import jax
import jax.numpy as jnp
from jax.experimental import pallas as pl
from jax.experimental.pallas import tpu as pltpu


def _rvq_body(emb_ref, cb_ref, stack_ref, hcsq_ref, out_ref, resid_ref):
    nq = pl.num_programs(0)
    q = pl.program_id(0)
    i = pl.program_id(1)
    tn = out_ref.shape[2]
    half = tn // 2
    k = cb_ref.shape[1]

    @pl.when(q == 0)
    def _():
        resid_ref[pl.ds(i * tn, tn), :] = emb_ref[pl.ds(i * tn, tn), :]

    def mm(a, b, contract_b):
        return jax.lax.dot_general(
            a, b, (((1,), (contract_b,)), ((), ())),
            preferred_element_type=jnp.float32,
        )

    rows = pl.ds(i * tn, tn)
    resid = resid_ref[rows, :]
    dots = mm(resid, cb_ref[0], 1)
    score = dots - hcsq_ref[q][None, :]
    maxv = jnp.max(score, axis=1, keepdims=True)
    eq = score == maxv
    iota = jax.lax.broadcasted_iota(jnp.int32, (tn, k), 1)
    idx = jnp.min(jnp.where(eq, iota, k), axis=1).astype(jnp.int32)
    out_ref[0, 0, :] = idx

    @pl.when(q < nq - 1)
    def _():
        oh = eq.astype(jnp.bfloat16)
        onehot3 = jnp.concatenate([oh, oh, oh], axis=1)
        quant = mm(onehot3, stack_ref[0], 0)
        resid_ref[rows, :] = resid - quant


def kernel(embedding, codebooks):
    n, d = embedding.shape
    nq, k, _ = codebooks.shape
    tn = min(1024, n)
    grid_n = n // tn

    half_csq = 0.5 * jnp.sum(codebooks * codebooks, axis=-1)

    def trunc_hi16(x):
        return jax.lax.bitcast_convert_type(
            jax.lax.bitcast_convert_type(x, jnp.uint32) & jnp.uint32(0xFFFF0000),
            jnp.float32)

    hi_v = trunc_hi16(codebooks)
    r1 = codebooks - hi_v
    mid_v = trunc_hi16(r1)
    lo_v = r1 - mid_v
    cb_stack = jnp.concatenate(
        [hi_v.astype(jnp.bfloat16), mid_v.astype(jnp.bfloat16),
         lo_v.astype(jnp.bfloat16)], axis=1)

    out = pl.pallas_call(
        _rvq_body,
        grid=(nq, grid_n),
        in_specs=[
            pl.BlockSpec((n, d), lambda q, i: (0, 0)),
            pl.BlockSpec((1, k, d), lambda q, i: (q, 0, 0)),
            pl.BlockSpec((1, 3 * k, d), lambda q, i: (q, 0, 0)),
            pl.BlockSpec((nq, k), lambda q, i: (0, 0)),
        ],
        out_specs=pl.BlockSpec((1, 1, tn), lambda q, i: (q, 0, i)),
        out_shape=jax.ShapeDtypeStruct((nq, 1, n), jnp.int32),
        scratch_shapes=[pltpu.VMEM((n, d), jnp.float32)],
    )(embedding, codebooks, cb_stack, half_csq)

    return jnp.transpose(out.reshape(nq, n))[None, :, :]


# speedup vs baseline: 1.0650x; 1.0650x over previous
import functools

import jax
import jax.numpy as jnp
from jax import lax
from jax.experimental import pallas as pl
from jax.experimental.pallas import tpu as pltpu
from jax.experimental.pallas import tpu_sc as plsc

_NC, _NS = 2, 16


def _sc_gather(cb, idx):
    k, d = cb.shape
    n = idx.shape[0]
    nw = _NC * _NS
    bw = n // nw
    mesh = plsc.VectorSubcoreMesh(core_axis_name="c", subcore_axis_name="s")

    @functools.partial(
        pl.kernel, mesh=mesh,
        out_type=jax.ShapeDtypeStruct((n, d), jnp.float32),
        scratch_types=[
            pltpu.VMEM((bw,), jnp.int32),
            pltpu.VMEM((bw, d), jnp.float32),
            pltpu.SemaphoreType.DMA,
        ],
    )
    def k_fn(cb_hbm, idx_hbm, out_hbm, idx_v, rows_v, sem):
        wid = lax.axis_index("s") * _NC + lax.axis_index("c")
        base = wid * bw
        pltpu.sync_copy(idx_hbm.at[pl.ds(base, bw)], idx_v)
        pltpu.async_copy(cb_hbm.at[idx_v], rows_v, sem).wait()
        pltpu.sync_copy(rows_v, out_hbm.at[pl.ds(base, bw)])

    return k_fn(cb, idx)


def _tc_body(has_quant, has_rout, *refs):
    if has_quant:
        resid_ref, quant_ref, cb_ref, hcsq_ref = refs[:4]
        outs = refs[4:]
    else:
        resid_ref, cb_ref, hcsq_ref = refs[:3]
        outs = refs[3:]
    idx_ref = outs[0]

    r = resid_ref[...]
    if has_quant:
        r = r - quant_ref[...]
    if has_rout:
        outs[1][...] = r
    dots = jax.lax.dot_general(
        r, cb_ref[...], (((1,), (1,)), ((), ())),
        preferred_element_type=jnp.float32,
    )
    score = dots - hcsq_ref[0][None, :]
    idx_ref[0, :] = jnp.argmax(score, axis=1).astype(jnp.int32)


def _tc_stage(resid, quant, cb, hcsq, want_rout):
    n, d = resid.shape
    k = cb.shape[0]
    tn = min(1024, n)
    grid = n // tn
    has_quant = quant is not None

    in_specs = [pl.BlockSpec((tn, d), lambda i: (i, 0))]
    args = [resid]
    if has_quant:
        in_specs.append(pl.BlockSpec((tn, d), lambda i: (i, 0)))
        args.append(quant)
    in_specs += [
        pl.BlockSpec((k, d), lambda i: (0, 0)),
        pl.BlockSpec((1, k), lambda i: (0, 0)),
    ]
    args += [cb, hcsq]

    out_specs = [pl.BlockSpec((1, tn), lambda i: (0, i))]
    out_shape = [jax.ShapeDtypeStruct((1, n), jnp.int32)]
    if want_rout:
        out_specs.append(pl.BlockSpec((tn, d), lambda i: (i, 0)))
        out_shape.append(jax.ShapeDtypeStruct((n, d), jnp.float32))

    res = pl.pallas_call(
        functools.partial(_tc_body, has_quant, want_rout),
        grid=(grid,),
        in_specs=in_specs,
        out_specs=out_specs,
        out_shape=out_shape,
    )(*args)
    return res if want_rout else (res[0], None)


def kernel(embedding, codebooks):
    n, d = embedding.shape
    nq, k, _ = codebooks.shape
    half_csq = 0.5 * jnp.sum(codebooks * codebooks, axis=-1)

    indices = []
    resid = embedding
    quant = None
    for q in range(nq):
        idx, rout = _tc_stage(resid, quant, codebooks[q],
                              half_csq[q][None, :], want_rout=0 < q < nq - 1)
        indices.append(idx[0])
        if q < nq - 1:
            quant = _sc_gather(codebooks[q], idx[0])
            if rout is not None:
                resid = rout
    return jnp.stack(indices, axis=-1)[None, :, :]


# speedup vs baseline: 1.1253x; 1.0566x over previous
import functools

import jax
import jax.numpy as jnp
from jax import lax
from jax.experimental import pallas as pl
from jax.experimental.pallas import tpu as pltpu
from jax.experimental.pallas import tpu_sc as plsc

_NC, _NS = 2, 16


def _sc_gather(cb, idx):
    k, d = cb.shape
    n = idx.shape[0]
    nw = _NC * _NS
    bw = n // nw
    mesh = plsc.VectorSubcoreMesh(core_axis_name="c", subcore_axis_name="s")

    @functools.partial(
        pl.kernel, mesh=mesh,
        out_type=jax.ShapeDtypeStruct((n, d), jnp.float32),
        scratch_types=[
            pltpu.VMEM((bw,), jnp.int32),
            pltpu.VMEM((bw, d), jnp.float32),
            pltpu.SemaphoreType.DMA,
        ],
    )
    def k_fn(cb_hbm, idx_hbm, out_hbm, idx_v, rows_v, sem):
        wid = lax.axis_index("s") * _NC + lax.axis_index("c")
        base = wid * bw
        pltpu.sync_copy(idx_hbm.at[pl.ds(base, bw)], idx_v)
        pltpu.async_copy(cb_hbm.at[idx_v], rows_v, sem).wait()
        pltpu.sync_copy(rows_v, out_hbm.at[pl.ds(base, bw)])

    return k_fn(cb, idx)


def _tc_body(has_quant, has_rout, *refs):
    if has_quant:
        resid_ref, quant_ref, cb_ref, hcsq_ref = refs[:4]
        outs = refs[4:]
    else:
        resid_ref, cb_ref, hcsq_ref = refs[:3]
        outs = refs[3:]
    idx_ref = outs[0]

    r = resid_ref[...]
    if has_quant:
        r = r - quant_ref[...]
    if has_rout:
        outs[1][...] = r
    dots = jax.lax.dot_general(
        r, cb_ref[...], (((1,), (1,)), ((), ())),
        preferred_element_type=jnp.float32,
    )
    score = dots - hcsq_ref[0][None, :]
    idx_ref[0, :] = jnp.argmax(score, axis=1).astype(jnp.int32)


def _tc_stage(resid, quant, cb, hcsq, want_rout):
    n, d = resid.shape
    k = cb.shape[0]
    tn = min(1024, n)
    grid = n // tn
    has_quant = quant is not None

    in_specs = [pl.BlockSpec((tn, d), lambda i: (i, 0))]
    args = [resid]
    if has_quant:
        in_specs.append(pl.BlockSpec((tn, d), lambda i: (i, 0)))
        args.append(quant)
    in_specs += [
        pl.BlockSpec((k, d), lambda i: (0, 0)),
        pl.BlockSpec((1, k), lambda i: (0, 0)),
    ]
    args += [cb, hcsq]

    out_specs = [pl.BlockSpec((1, tn), lambda i: (0, i))]
    out_shape = [jax.ShapeDtypeStruct((1, n), jnp.int32)]
    if want_rout:
        out_specs.append(pl.BlockSpec((tn, d), lambda i: (i, 0)))
        out_shape.append(jax.ShapeDtypeStruct((n, d), jnp.float32))

    res = pl.pallas_call(
        functools.partial(_tc_body, has_quant, want_rout),
        grid=(grid,),
        in_specs=in_specs,
        out_specs=out_specs,
        out_shape=out_shape,
    )(*args)
    return res if want_rout else (res[0], None)


def kernel(embedding, codebooks):
    n, d = embedding.shape
    nq, k, _ = codebooks.shape
    half_csq = 0.5 * jnp.sum(codebooks * codebooks, axis=-1)

    nh = n // 2
    half_out = []
    for h in range(2):
        emb_h = jax.lax.slice_in_dim(embedding, h * nh, (h + 1) * nh, axis=0)
        indices = []
        resid = emb_h
        quant = None
        for q in range(nq):
            idx, rout = _tc_stage(resid, quant, codebooks[q],
                                  half_csq[q][None, :],
                                  want_rout=0 < q < nq - 1)
            indices.append(idx[0])
            if q < nq - 1:
                quant = _sc_gather(codebooks[q], idx[0])
                if rout is not None:
                    resid = rout
        half_out.append(jnp.stack(indices, axis=-1))
    return jnp.concatenate(half_out, axis=0)[None, :, :]


# speedup vs baseline: 1.1260x; 1.0007x over previous
import jax
import jax.numpy as jnp
from jax.experimental import pallas as pl
from jax.experimental.pallas import tpu as pltpu


def _rvq_body(emb_ref, cb_ref, stack_ref, hcsq_ref, out_ref, resid_ref):
    nq = pl.num_programs(0)
    q = pl.program_id(0)
    i = pl.program_id(1)
    tn = out_ref.shape[2]
    half = tn // 2
    k = cb_ref.shape[1]

    @pl.when(q == 0)
    def _():
        resid_ref[pl.ds(i * tn, tn), :] = emb_ref[pl.ds(i * tn, tn), :]

    def mm(a, b, contract_b):
        return jax.lax.dot_general(
            a, b, (((1,), (contract_b,)), ((), ())),
            preferred_element_type=jnp.float32,
        )

    rows = pl.ds(i * tn, tn)
    resid = resid_ref[rows, :]
    dots = mm(resid, cb_ref[0], 1)
    score = dots - hcsq_ref[q][None, :]
    idx = jnp.argmax(score, axis=1).astype(jnp.int32)
    out_ref[0, 0, :] = idx

    @pl.when(q < nq - 1)
    def _():
        iota = jax.lax.broadcasted_iota(jnp.int32, (tn, k), 1)
        oh = (iota == idx[:, None]).astype(jnp.bfloat16)
        onehot3 = jnp.concatenate([oh, oh, oh], axis=1)
        quant = mm(onehot3, stack_ref[0], 0)
        resid_ref[rows, :] = resid - quant


def kernel(embedding, codebooks):
    n, d = embedding.shape
    nq, k, _ = codebooks.shape
    tn = min(1024, n)
    grid_n = n // tn

    half_csq = 0.5 * jnp.sum(codebooks * codebooks, axis=-1)

    def trunc_hi16(x):
        return jax.lax.bitcast_convert_type(
            jax.lax.bitcast_convert_type(x, jnp.uint32) & jnp.uint32(0xFFFF0000),
            jnp.float32)

    hi_v = trunc_hi16(codebooks)
    r1 = codebooks - hi_v
    mid_v = trunc_hi16(r1)
    lo_v = r1 - mid_v
    cb_stack = jnp.concatenate(
        [hi_v.astype(jnp.bfloat16), mid_v.astype(jnp.bfloat16),
         lo_v.astype(jnp.bfloat16)], axis=1)

    out = pl.pallas_call(
        _rvq_body,
        grid=(nq, grid_n),
        in_specs=[
            pl.BlockSpec((n, d), lambda q, i: (0, 0)),
            pl.BlockSpec((1, k, d), lambda q, i: (q, 0, 0)),
            pl.BlockSpec((1, 3 * k, d), lambda q, i: (q, 0, 0)),
            pl.BlockSpec((nq, k), lambda q, i: (0, 0)),
        ],
        out_specs=pl.BlockSpec((1, 1, tn), lambda q, i: (q, 0, i)),
        out_shape=jax.ShapeDtypeStruct((nq, 1, n), jnp.int32),
        scratch_shapes=[pltpu.VMEM((n, d), jnp.float32)],
    )(embedding, codebooks, cb_stack, half_csq)

    return jnp.transpose(out.reshape(nq, n))[None, :, :]
